# baseline (speedup 1.0000x reference)
import jax
import jax.numpy as jnp
from jax.experimental import pallas as pl
from jax.experimental.pallas import tpu as pltpu
from jax.experimental.pallas import tpu_sc as plsc

RHO = 1.0
BETA = 0.5
TAU = 0.1
EPS = 1e-9

_N_TOK = 2048
_K = 32
_D = 768
_H = _D // 2
_B = 8
_SQW = 128


def _rne_hi16(t):
    return t + jnp.int32(0x7FFF) + ((t >> 16) & jnp.int32(1))


def _prep_body(m_ref, pk_ref, sq_ref):
    M = m_ref[...]
    t = jax.lax.bitcast_convert_type(M, jnp.int32)
    t1 = _rne_hi16(t[:, :_H])
    t2 = _rne_hi16(t[:, _H:])
    pk_ref[...] = ((t1 >> 16) & jnp.int32(0xFFFF)) | (
        t2 & jnp.int32(-65536))
    sq_ref[...] = jnp.broadcast_to(
        jnp.sum(M * M, axis=1, keepdims=True), (m_ref.shape[0], _SQW))


def _bank_prep(M):
    nrows = M.shape[0]
    blk = 512
    return pl.pallas_call(
        _prep_body,
        grid=(nrows // blk,),
        in_specs=[pl.BlockSpec((blk, M.shape[1]), lambda i: (i, 0))],
        out_specs=[pl.BlockSpec((blk, _H), lambda i: (i, 0)),
                   pl.BlockSpec((blk, _SQW), lambda i: (i, 0))],
        out_shape=[jax.ShapeDtypeStruct((nrows, _H), jnp.int32),
                   jax.ShapeDtypeStruct((nrows, _SQW), jnp.float32)],
    )(M)


def _sc_gather(bank, idx_flat, window):
    num_idx = idx_flat.shape[0]
    d = bank.shape[1]
    indices = idx_flat.reshape(num_idx // window, window)
    mesh = plsc.VectorSubcoreMesh(core_axis_name="core",
                                  subcore_axis_name="subcore")

    @pl.kernel(out_type=jax.ShapeDtypeStruct((num_idx, d), bank.dtype),
               mesh=mesh)
    def gather_kernel(m_hbm, i_hbm, o_hbm):
        def body(i_vmem, o_vmem):
            pltpu.sync_copy(m_hbm.at[i_vmem.at[0]], o_vmem)

        pltpu.emit_pipeline(
            body,
            grid=(num_idx // window,),
            in_specs=[pl.BlockSpec((1, window), lambda i: (i, 0))],
            out_specs=[pl.BlockSpec((window, d), lambda i: (i, 0))],
            core_axis_name=("core", "subcore"),
            dimension_semantics=(pltpu.PARALLEL,),
        )(i_hbm, o_hbm)

    return gather_kernel(bank, indices)


def _tc_body(y_ref, p_ref, lam_ref, t_ref, sq_ref, pnew_ref, lamnew_ref):
    W = t_ref[...]
    Tlo = jax.lax.bitcast_convert_type(W << 16, jnp.float32)
    Thi = jax.lax.bitcast_convert_type(W & jnp.int32(-65536), jnp.float32)
    P3 = p_ref[...]
    Y = y_ref[...]
    Lam = lam_ref[...]
    sq = sq_ref[...][:, :, :1]
    Ylo, Yhi = Y[:, :_H], Y[:, _H:]
    Llo, Lhi = Lam[:, :_H], Lam[:, _H:]

    S = jnp.sum(P3, axis=0, keepdims=True)
    Pn3 = P3 / (S + EPS)
    Pnb = Pn3.astype(jnp.bfloat16).astype(jnp.float32)
    yfp_lo = jnp.sum(Tlo * Pnb, axis=0)
    yfp_hi = jnp.sum(Thi * Pnb, axis=0)
    Xib_lo = (Llo + RHO * (Ylo - yfp_lo)
              ).astype(jnp.bfloat16).astype(jnp.float32)
    Xib_hi = (Lhi + RHO * (Yhi - yfp_hi)
              ).astype(jnp.bfloat16).astype(jnp.float32)
    yp_lo = jnp.sum(Tlo * Pn3, axis=0)
    yp_hi = jnp.sum(Thi * Pn3, axis=0)
    Vlo = (2.0 * TAU) * yp_lo - BETA * Xib_lo
    Vhi = (2.0 * TAU) * yp_hi - BETA * Xib_hi
    ws = (jnp.sum(Tlo * Vlo[None, :, :], axis=2, keepdims=True)
          + jnp.sum(Thi * Vhi[None, :, :], axis=2, keepdims=True))
    logits = jnp.log(Pn3 + EPS) - TAU * sq + ws
    m = jnp.max(logits, axis=0, keepdims=True)
    e = jnp.exp(logits - m)
    Pnew3 = e / jnp.sum(e, axis=0, keepdims=True)
    Pnewb = Pnew3.astype(jnp.bfloat16).astype(jnp.float32)
    yfp2_lo = jnp.sum(Tlo * Pnewb, axis=0)
    yfp2_hi = jnp.sum(Thi * Pnewb, axis=0)
    pnew_ref[...] = Pnew3
    lamnew_ref[:, :_H] = Llo + RHO * (Ylo - yfp2_lo)
    lamnew_ref[:, _H:] = Lhi + RHO * (Yhi - yfp2_hi)


def _tc_compute(Y, P, Lam, T, Tsq):
    n, d = Y.shape
    k = P.shape[1]
    grid = (n // _B,)
    pnew_t, lam_new = pl.pallas_call(
        _tc_body,
        grid=grid,
        in_specs=[
            pl.BlockSpec((_B, d), lambda i: (i, 0)),
            pl.BlockSpec((k, _B, 1), lambda i: (0, i, 0)),
            pl.BlockSpec((_B, d), lambda i: (i, 0)),
            pl.BlockSpec((k, _B, _H), lambda i: (0, i, 0)),
            pl.BlockSpec((k, _B, _SQW), lambda i: (0, i, 0)),
        ],
        out_specs=[
            pl.BlockSpec((k, _B, 1), lambda i: (0, i, 0)),
            pl.BlockSpec((_B, d), lambda i: (i, 0)),
        ],
        out_shape=[
            jax.ShapeDtypeStruct((k, n, 1), jnp.float32),
            jax.ShapeDtypeStruct((n, d), jnp.float32),
        ],
    )(Y, P.T.reshape(k, n, 1), Lam, T, Tsq)
    return pnew_t.reshape(k, n).T, lam_new


_NC = 4


def kernel(Y, P, Lam, M, Kset):
    n, k = Kset.shape
    nc = n // _NC
    B32, sqM = _bank_prep(M)
    gathered = []
    for c in range(_NC):
        lo, hi = c * nc, (c + 1) * nc
        idx = Kset[lo:hi].T.reshape(nc * k)
        T32 = _sc_gather(B32, idx, 128)
        Tsq = _sc_gather(sqM, idx, 256)
        gathered.append((T32, Tsq))
    p_parts, l_parts = [], []
    for c in range(_NC):
        lo, hi = c * nc, (c + 1) * nc
        T32, Tsq = gathered[c]
        p_c, l_c = _tc_compute(Y[lo:hi], P[lo:hi], Lam[lo:hi],
                               T32.reshape(k, nc, _H),
                               Tsq.reshape(k, nc, _SQW))
        p_parts.append(p_c)
        l_parts.append(l_c)
    return (jnp.concatenate(p_parts, axis=0),
            jnp.concatenate(l_parts, axis=0))


# speedup vs baseline: 1.3974x; 1.3974x over previous
import jax
import jax.numpy as jnp
from jax.experimental import pallas as pl
from jax.experimental.pallas import tpu as pltpu
from jax.experimental.pallas import tpu_sc as plsc

RHO = 1.0
BETA = 0.5
TAU = 0.1
EPS = 1e-9

_N_TOK = 2048
_K = 32
_D = 768
_H = _D // 2
_B = 32
_SQW = 128


def _rne_hi16(t):
    return t + jnp.int32(0x7FFF) + ((t >> 16) & jnp.int32(1))


def _prep_body(m_ref, pk_ref, sq_ref):
    M = m_ref[...]
    t = jax.lax.bitcast_convert_type(M, jnp.int32)
    t1 = _rne_hi16(t[:, :_H])
    t2 = _rne_hi16(t[:, _H:])
    pk_ref[...] = ((t1 >> 16) & jnp.int32(0xFFFF)) | (
        t2 & jnp.int32(-65536))
    sq_ref[...] = jnp.broadcast_to(
        jnp.sum(M * M, axis=1, keepdims=True), (m_ref.shape[0], _SQW))


def _bank_prep(M):
    nrows = M.shape[0]
    blk = 512
    return pl.pallas_call(
        _prep_body,
        grid=(nrows // blk,),
        in_specs=[pl.BlockSpec((blk, M.shape[1]), lambda i: (i, 0))],
        out_specs=[pl.BlockSpec((blk, _H), lambda i: (i, 0)),
                   pl.BlockSpec((blk, _SQW), lambda i: (i, 0))],
        out_shape=[jax.ShapeDtypeStruct((nrows, _H), jnp.int32),
                   jax.ShapeDtypeStruct((nrows, _SQW), jnp.float32)],
    )(M)


def _sc_gather(bank, idx_flat, window):
    num_idx = idx_flat.shape[0]
    d = bank.shape[1]
    indices = idx_flat.reshape(num_idx // window, window)
    mesh = plsc.VectorSubcoreMesh(core_axis_name="core",
                                  subcore_axis_name="subcore")

    @pl.kernel(out_type=jax.ShapeDtypeStruct((num_idx, d), bank.dtype),
               mesh=mesh)
    def gather_kernel(m_hbm, i_hbm, o_hbm):
        def body(i_vmem, o_vmem):
            pltpu.sync_copy(m_hbm.at[i_vmem.at[0]], o_vmem)

        pltpu.emit_pipeline(
            body,
            grid=(num_idx // window,),
            in_specs=[pl.BlockSpec((1, window), lambda i: (i, 0))],
            out_specs=[pl.BlockSpec((window, d), lambda i: (i, 0))],
            core_axis_name=("core", "subcore"),
            dimension_semantics=(pltpu.PARALLEL,),
        )(i_hbm, o_hbm)

    return gather_kernel(bank, indices)


def _tc_body(y_ref, p_ref, lam_ref, t_ref, sq_ref, pnew_ref, lamnew_ref):
    W = t_ref[...]
    Tlo = jax.lax.bitcast_convert_type(W << 16, jnp.float32)
    Thi = jax.lax.bitcast_convert_type(W & jnp.int32(-65536), jnp.float32)
    P3 = p_ref[...]
    Y = y_ref[...]
    Lam = lam_ref[...]
    sq = sq_ref[...][:, :, :1]
    Ylo, Yhi = Y[:, :_H], Y[:, _H:]
    Llo, Lhi = Lam[:, :_H], Lam[:, _H:]

    S = jnp.sum(P3, axis=0, keepdims=True)
    Pn3 = P3 / (S + EPS)
    Pnb = Pn3.astype(jnp.bfloat16).astype(jnp.float32)
    yfp_lo = jnp.sum(Tlo * Pnb, axis=0)
    yfp_hi = jnp.sum(Thi * Pnb, axis=0)
    Xib_lo = (Llo + RHO * (Ylo - yfp_lo)
              ).astype(jnp.bfloat16).astype(jnp.float32)
    Xib_hi = (Lhi + RHO * (Yhi - yfp_hi)
              ).astype(jnp.bfloat16).astype(jnp.float32)
    yp_lo = jnp.sum(Tlo * Pn3, axis=0)
    yp_hi = jnp.sum(Thi * Pn3, axis=0)
    Vlo = (2.0 * TAU) * yp_lo - BETA * Xib_lo
    Vhi = (2.0 * TAU) * yp_hi - BETA * Xib_hi
    ws = (jnp.sum(Tlo * Vlo[None, :, :], axis=2, keepdims=True)
          + jnp.sum(Thi * Vhi[None, :, :], axis=2, keepdims=True))
    logits = jnp.log(Pn3 + EPS) - TAU * sq + ws
    m = jnp.max(logits, axis=0, keepdims=True)
    e = jnp.exp(logits - m)
    Pnew3 = e / jnp.sum(e, axis=0, keepdims=True)
    Pnewb = Pnew3.astype(jnp.bfloat16).astype(jnp.float32)
    yfp2_lo = jnp.sum(Tlo * Pnewb, axis=0)
    yfp2_hi = jnp.sum(Thi * Pnewb, axis=0)
    pnew_ref[...] = Pnew3
    lamnew_ref[:, :_H] = Llo + RHO * (Ylo - yfp2_lo)
    lamnew_ref[:, _H:] = Lhi + RHO * (Yhi - yfp2_hi)


def _tc_compute(Y, PT, Lam, T, Tsq):
    n, d = Y.shape
    k = PT.shape[0]
    grid = (n // _B,)
    pnew_t, lam_new = pl.pallas_call(
        _tc_body,
        grid=grid,
        in_specs=[
            pl.BlockSpec((_B, d), lambda i: (i, 0)),
            pl.BlockSpec((k, _B, 1), lambda i: (0, i, 0)),
            pl.BlockSpec((_B, d), lambda i: (i, 0)),
            pl.BlockSpec((k, _B, _H), lambda i: (0, i, 0)),
            pl.BlockSpec((k, _B, _SQW), lambda i: (0, i, 0)),
        ],
        out_specs=[
            pl.BlockSpec((k, _B, 1), lambda i: (0, i, 0)),
            pl.BlockSpec((_B, d), lambda i: (i, 0)),
        ],
        out_shape=[
            jax.ShapeDtypeStruct((k, n, 1), jnp.float32),
            jax.ShapeDtypeStruct((n, d), jnp.float32),
        ],
    )(Y, PT, Lam, T, Tsq)
    return pnew_t, lam_new


_NC = 4


def kernel(Y, P, Lam, M, Kset):
    n, k = Kset.shape
    nc = n // _NC
    B32, sqM = _bank_prep(M)
    KT = Kset.T
    PT = P.T.reshape(k, n, 1)
    gathered = []
    for c in range(_NC):
        lo, hi = c * nc, (c + 1) * nc
        idx = KT[:, lo:hi].reshape(nc * k)
        T32 = _sc_gather(B32, idx, 128)
        Tsq = _sc_gather(sqM, idx, 256)
        gathered.append((T32, Tsq))
    p_parts, l_parts = [], []
    for c in range(_NC):
        lo, hi = c * nc, (c + 1) * nc
        T32, Tsq = gathered[c]
        p_c, l_c = _tc_compute(Y[lo:hi], PT[:, lo:hi], Lam[lo:hi],
                               T32.reshape(k, nc, _H),
                               Tsq.reshape(k, nc, _SQW))
        p_parts.append(p_c)
        l_parts.append(l_c)
    P_new = jnp.concatenate(p_parts, axis=1).reshape(k, n).T
    Lam_new = jnp.concatenate(l_parts, axis=0)
    return (P_new, Lam_new)


# speedup vs baseline: 1.4401x; 1.0306x over previous
import jax
import jax.numpy as jnp
from jax.experimental import pallas as pl
from jax.experimental.pallas import tpu as pltpu
from jax.experimental.pallas import tpu_sc as plsc

RHO = 1.0
BETA = 0.5
TAU = 0.1
EPS = 1e-9

_N_TOK = 2048
_K = 32
_D = 768
_H = _D // 2
_B = 32
_SQW = 128


def _rne_hi16(t):
    return t + jnp.int32(0x7FFF) + ((t >> 16) & jnp.int32(1))


def _prep_body(m_ref, pk_ref, sq_ref):
    M = m_ref[...]
    t = jax.lax.bitcast_convert_type(M, jnp.int32)
    t1 = _rne_hi16(t[:, :_H])
    t2 = _rne_hi16(t[:, _H:])
    pk_ref[...] = ((t1 >> 16) & jnp.int32(0xFFFF)) | (
        t2 & jnp.int32(-65536))
    sq_ref[...] = jnp.broadcast_to(
        jnp.sum(M * M, axis=1, keepdims=True), (m_ref.shape[0], _SQW))


def _bank_prep(M):
    nrows = M.shape[0]
    blk = 512
    return pl.pallas_call(
        _prep_body,
        grid=(nrows // blk,),
        in_specs=[pl.BlockSpec((blk, M.shape[1]), lambda i: (i, 0))],
        out_specs=[pl.BlockSpec((blk, _H), lambda i: (i, 0)),
                   pl.BlockSpec((blk, _SQW), lambda i: (i, 0))],
        out_shape=[jax.ShapeDtypeStruct((nrows, _H), jnp.int32),
                   jax.ShapeDtypeStruct((nrows, _SQW), jnp.float32)],
    )(M)


def _sc_gather(bank, idx_flat, window):
    num_idx = idx_flat.shape[0]
    d = bank.shape[1]
    indices = idx_flat.reshape(num_idx // window, window)
    mesh = plsc.VectorSubcoreMesh(core_axis_name="core",
                                  subcore_axis_name="subcore")

    @pl.kernel(out_type=jax.ShapeDtypeStruct((num_idx, d), bank.dtype),
               mesh=mesh)
    def gather_kernel(m_hbm, i_hbm, o_hbm):
        def body(i_vmem, o_vmem):
            pltpu.sync_copy(m_hbm.at[i_vmem.at[0]], o_vmem)

        pltpu.emit_pipeline(
            body,
            grid=(num_idx // window,),
            in_specs=[pl.BlockSpec((1, window), lambda i: (i, 0))],
            out_specs=[pl.BlockSpec((window, d), lambda i: (i, 0))],
            core_axis_name=("core", "subcore"),
            dimension_semantics=(pltpu.PARALLEL,),
        )(i_hbm, o_hbm)

    return gather_kernel(bank, indices)


def _tc_body(y_ref, p_ref, lam_ref, t_ref, sq_ref, pnew_ref, lamnew_ref):
    W = t_ref[...]
    Tlo = jax.lax.bitcast_convert_type(W << 16, jnp.float32)
    Thi = jax.lax.bitcast_convert_type(W & jnp.int32(-65536), jnp.float32)
    P3 = p_ref[...]
    Y = y_ref[...]
    Lam = lam_ref[...]
    sq = sq_ref[...][:, :, :1]
    Ylo, Yhi = Y[:, :_H], Y[:, _H:]
    Llo, Lhi = Lam[:, :_H], Lam[:, _H:]

    S = jnp.sum(P3, axis=0, keepdims=True)
    Pn3 = P3 / (S + EPS)
    Pnb = Pn3.astype(jnp.bfloat16).astype(jnp.float32)
    yfp_lo = jnp.sum(Tlo * Pnb, axis=0)
    yfp_hi = jnp.sum(Thi * Pnb, axis=0)
    Xib_lo = (Llo + RHO * (Ylo - yfp_lo)
              ).astype(jnp.bfloat16).astype(jnp.float32)
    Xib_hi = (Lhi + RHO * (Yhi - yfp_hi)
              ).astype(jnp.bfloat16).astype(jnp.float32)
    yp_lo = jnp.sum(Tlo * Pn3, axis=0)
    yp_hi = jnp.sum(Thi * Pn3, axis=0)
    Vlo = (2.0 * TAU) * yp_lo - BETA * Xib_lo
    Vhi = (2.0 * TAU) * yp_hi - BETA * Xib_hi
    ws = (jnp.sum(Tlo * Vlo[None, :, :], axis=2, keepdims=True)
          + jnp.sum(Thi * Vhi[None, :, :], axis=2, keepdims=True))
    logits = jnp.log(Pn3 + EPS) - TAU * sq + ws
    m = jnp.max(logits, axis=0, keepdims=True)
    e = jnp.exp(logits - m)
    Pnew3 = e / jnp.sum(e, axis=0, keepdims=True)
    Pnewb = Pnew3.astype(jnp.bfloat16).astype(jnp.float32)
    yfp2_lo = jnp.sum(Tlo * Pnewb, axis=0)
    yfp2_hi = jnp.sum(Thi * Pnewb, axis=0)
    pnew_ref[...] = Pnew3
    lamnew_ref[:, :_H] = Llo + RHO * (Ylo - yfp2_lo)
    lamnew_ref[:, _H:] = Lhi + RHO * (Yhi - yfp2_hi)


def _tc_compute(Y, PT, Lam, T, Tsq, co, nc):
    n, d = Y.shape
    k = PT.shape[0]
    grid = (nc // _B,)
    pnew_t, lam_new = pl.pallas_call(
        _tc_body,
        grid=grid,
        in_specs=[
            pl.BlockSpec((_B, d), lambda i: (i + co, 0)),
            pl.BlockSpec((k, _B, 1), lambda i: (0, i + co, 0)),
            pl.BlockSpec((_B, d), lambda i: (i + co, 0)),
            pl.BlockSpec((k, _B, _H), lambda i: (0, i, 0)),
            pl.BlockSpec((k, _B, _SQW), lambda i: (0, i, 0)),
        ],
        out_specs=[
            pl.BlockSpec((k, _B, 1), lambda i: (0, i, 0)),
            pl.BlockSpec((_B, d), lambda i: (i, 0)),
        ],
        out_shape=[
            jax.ShapeDtypeStruct((k, nc, 1), jnp.float32),
            jax.ShapeDtypeStruct((nc, d), jnp.float32),
        ],
    )(Y, PT, Lam, T, Tsq)
    return pnew_t, lam_new


_CHUNKS = (128, 640, 640, 640)


def kernel(Y, P, Lam, M, Kset):
    n, k = Kset.shape
    B32, sqM = _bank_prep(M)
    KT = Kset.T
    PT = P.T.reshape(k, n, 1)
    gathered, lo = [], 0
    for nc in _CHUNKS:
        idx = KT[:, lo:lo + nc].reshape(nc * k)
        T32 = _sc_gather(B32, idx, 128)
        Tsq = _sc_gather(sqM, idx, 128)
        gathered.append((T32, Tsq))
        lo += nc
    p_parts, l_parts, lo = [], [], 0
    for c, nc in enumerate(_CHUNKS):
        T32, Tsq = gathered[c]
        p_c, l_c = _tc_compute(Y, PT, Lam, T32.reshape(k, nc, _H),
                               Tsq.reshape(k, nc, _SQW), lo // _B, nc)
        p_parts.append(p_c)
        l_parts.append(l_c)
        lo += nc
    P_new = jnp.concatenate(p_parts, axis=1).reshape(k, n).T
    Lam_new = jnp.concatenate(l_parts, axis=0)
    return (P_new, Lam_new)
